# baseline (speedup 1.0000x reference)
import functools

import jax
import jax.numpy as jnp
from jax import lax
from jax.experimental import pallas as pl
from jax.experimental.pallas import tpu as pltpu
from jax.experimental.pallas import tpu_sc as plsc

_NUM_CORES = 2
_NUM_SUBCORES = 16
_NUM_WORKERS = _NUM_CORES * _NUM_SUBCORES
_LANES = 16

_ROWS = 32768
_CH = 2048

_BLOCK_ROWS = 16
_BLOCK_ELEMS = _BLOCK_ROWS * _CH
_ROWS_PER_WORKER = _ROWS // _NUM_WORKERS
_BLOCKS_PER_WORKER = _ROWS_PER_WORKER // _BLOCK_ROWS
_CHUNKS = _CH // _LANES


def _permute_body(x_hbm, p_hbm, out_hbm, p_v, in_v, out_v, in_sem, out_sem):
  wid = lax.axis_index("s") * _NUM_CORES + lax.axis_index("c")
  base = wid * _ROWS_PER_WORKER * _CH

  pltpu.sync_copy(p_hbm, p_v)

  def block_body(b, carry):
    elem0 = base + b * _BLOCK_ELEMS
    pltpu.async_copy(
        x_hbm.at[pl.ds(elem0, _BLOCK_ELEMS)], in_v, in_sem
    ).wait()

    def chunk_body(j, carry2):
      col0 = j * _LANES
      idx = p_v[pl.ds(col0, _LANES)]
      for r in range(_BLOCK_ROWS):
        vals = plsc.load_gather(in_v, [idx + (r * _CH)])
        out_v[pl.ds(r * _CH + col0, _LANES)] = vals
      return carry2

    lax.fori_loop(0, _CHUNKS, chunk_body, 0, unroll=2)

    pltpu.async_copy(
        out_v, out_hbm.at[pl.ds(elem0, _BLOCK_ELEMS)], out_sem
    ).wait()
    return carry

  lax.fori_loop(0, _BLOCKS_PER_WORKER, block_body, 0)


@jax.jit
def _permute(x_flat, p32):
  mesh = plsc.VectorSubcoreMesh(
      core_axis_name="c", subcore_axis_name="s",
      num_cores=_NUM_CORES, num_subcores=_NUM_SUBCORES,
  )
  kern = pl.kernel(
      _permute_body,
      out_type=jax.ShapeDtypeStruct((_ROWS * _CH,), jnp.float32),
      mesh=mesh,
      compiler_params=pltpu.CompilerParams(needs_layout_passes=False),
      scratch_types=[
          pltpu.VMEM((_CH,), jnp.int32),
          pltpu.VMEM((_BLOCK_ELEMS,), jnp.float32),
          pltpu.VMEM((_BLOCK_ELEMS,), jnp.float32),
          pltpu.SemaphoreType.DMA,
          pltpu.SemaphoreType.DMA,
      ],
  )
  return kern(x_flat, p32)


def kernel(x, p):
  p32 = p.astype(jnp.int32)
  out = _permute(x.reshape(-1), p32)
  return (out.reshape(_ROWS, _CH), 0)
